# baseline (speedup 1.0000x reference)
import functools

import jax
import jax.numpy as jnp
import numpy as np
from jax import lax
from jax.experimental import pallas as pl
from jax.experimental.pallas import tpu as pltpu
from jax.experimental.pallas import tpu_sc as plsc

RHO_NUM = 100
B = 16384
D = 11

_TC_BLK = 4096
_TC_GRID = B // _TC_BLK
_ROWS_PER_W = B // 32
_GROUPS = _ROWS_PER_W // 16
_IND_N = _TC_GRID * 128


def _bucket_thresholds():
    c = np.float32(1.0 / RHO_NUM)
    t = np.array([np.float32(0.01 * i) for i in range(1, RHO_NUM)], np.float32)

    def bucket_id(s):
        r = np.sqrt(np.float32(s), dtype=np.float32)
        k = np.round(np.float32(r / c)).astype(np.float32)
        return int(np.sum(np.float32(k * c) >= t))

    out = np.empty(RHO_NUM - 1, np.float32)
    for j in range(1, RHO_NUM):
        lo, hi = 0, int(np.array(1e8, np.float32).view(np.uint32))
        while lo < hi:
            mid = (lo + hi) // 2
            if bucket_id(np.array(mid, np.uint32).view(np.float32)) >= j:
                hi = mid
            else:
                lo = mid + 1
        out[j - 1] = np.array(lo, np.uint32).view(np.float32)
    return out


_S_LIST = [float(v) for v in _bucket_thresholds()]


def _tc_body(z1_ref, z2_ref, gidx_ref, msk_ref, ind_ref, s_scr, m_scr):
    i = pl.program_id(0)
    dx = z1_ref[:, D - 2] - z2_ref[:, D - 2]
    dy = z1_ref[:, D - 1] - z2_ref[:, D - 1]
    s = dx * dx + dy * dy
    mask = (z1_ref[:, 0] > 0.0) & (z2_ref[:, 0] > 0.0)
    sl = _TC_BLK // 128
    s_scr[...] = s.reshape(sl, 128)
    m_scr[...] = jnp.where(mask, 1.0, 0.0).reshape(sl, 128)
    s8 = s_scr[...]
    bid8 = jnp.zeros((sl, 128), jnp.int32)
    for thr in _S_LIST:
        bid8 = bid8 + (s8 >= thr).astype(jnp.int32)
    n8 = (lax.broadcasted_iota(jnp.int32, (sl, 128), 0) * 128
          + lax.broadcasted_iota(jnp.int32, (sl, 128), 1))
    gidx_ref[...] = ((i * _TC_BLK + n8) * RHO_NUM + bid8).reshape(1, sl, 128)
    msk_ref[...] = m_scr[...].reshape(1, sl, 128)
    ne = jnp.max(jnp.where(z1_ref[...] != z2_ref[...], 1.0, 0.0))
    ind_ref[...] = jnp.full((1, 1, 128), ne, jnp.float32)


def _tc_stage(z_1, z_2):
    sl = _TC_BLK // 128
    return pl.pallas_call(
        _tc_body,
        grid=(_TC_GRID,),
        in_specs=[
            pl.BlockSpec((_TC_BLK, D), lambda i: (i, 0)),
            pl.BlockSpec((_TC_BLK, D), lambda i: (i, 0)),
        ],
        out_specs=[
            pl.BlockSpec((1, sl, 128), lambda i: (i, 0, 0)),
            pl.BlockSpec((1, sl, 128), lambda i: (i, 0, 0)),
            pl.BlockSpec((1, 1, 128), lambda i: (i, 0, 0)),
        ],
        out_shape=[
            jax.ShapeDtypeStruct((_TC_GRID, sl, 128), jnp.int32),
            jax.ShapeDtypeStruct((_TC_GRID, sl, 128), jnp.float32),
            jax.ShapeDtypeStruct((_TC_GRID, 1, 128), jnp.float32),
        ],
        scratch_shapes=[
            pltpu.VMEM((sl, 128), jnp.float32),
            pltpu.VMEM((sl, 128), jnp.float32),
        ],
    )(z_1, z_2)


def _sc_body(dg_hbm, gidx_hbm, msk_hbm, ind_hbm, sat_hbm,
             idxv, maskv, valv, indv, sem):
    wid = lax.axis_index("s") * 2 + lax.axis_index("c")
    rbase = wid * _ROWS_PER_W

    pltpu.sync_copy(gidx_hbm.at[pl.ds(rbase, _ROWS_PER_W)], idxv)
    pltpu.sync_copy(msk_hbm.at[pl.ds(rbase, _ROWS_PER_W)], maskv)
    pltpu.sync_copy(ind_hbm, indv)

    copies = [
        pltpu.async_copy(
            dg_hbm.at[idxv.at[pl.ds(i * 128, 128)]],
            valv.at[pl.ds(i * 128, 128)],
            sem,
        )
        for i in range(_ROWS_PER_W // 128)
    ]

    ne = jnp.zeros((16,), jnp.float32)
    for k in range(_IND_N // 16):
        ne = jnp.maximum(ne, indv[pl.ds(k * 16, 16)])
    gate = jnp.where(jnp.max(ne) > 0.0, 1.0, 0.0)

    for c in copies:
        c.wait()

    for g in range(_GROUPS):
        slc = pl.ds(g * 16, 16)
        valv[slc] = valv[slc] * maskv[slc] * gate
    pltpu.sync_copy(valv, sat_hbm.at[pl.ds(rbase, _ROWS_PER_W)])


_sc_fn = functools.partial(
    pl.kernel,
    mesh=plsc.VectorSubcoreMesh(core_axis_name="c", subcore_axis_name="s"),
    compiler_params=pltpu.CompilerParams(needs_layout_passes=False),
    out_type=jax.ShapeDtypeStruct((B,), jnp.float32),
    scratch_types=[
        pltpu.VMEM((_ROWS_PER_W,), jnp.int32),
        pltpu.VMEM((_ROWS_PER_W,), jnp.float32),
        pltpu.VMEM((_ROWS_PER_W,), jnp.float32),
        pltpu.VMEM((_IND_N,), jnp.float32),
        pltpu.SemaphoreType.DMA,
    ],
)(_sc_body)


def kernel(z_1, z_2, dist_grade, img, given_param):
    gidx, msk, ind = _tc_stage(z_1, z_2)
    return _sc_fn(dist_grade.reshape(-1), gidx.reshape(-1),
                  msk.reshape(-1), ind.reshape(-1))


# speedup vs baseline: 1.1148x; 1.1148x over previous
import functools

import jax
import jax.numpy as jnp
import numpy as np
from jax import lax
from jax.experimental import pallas as pl
from jax.experimental.pallas import tpu as pltpu
from jax.experimental.pallas import tpu_sc as plsc

RHO_NUM = 100
B = 16384
D = 11

_TC_BLK = 4096
_TC_GRID = B // _TC_BLK
_ROWS_PER_W = B // 32
_GROUPS = _ROWS_PER_W // 16
_IND_N = _TC_GRID * 128


def _bucket_thresholds():
    c = np.float32(1.0 / RHO_NUM)
    t = np.array([np.float32(0.01 * i) for i in range(1, RHO_NUM)], np.float32)

    def bucket_id(s):
        r = np.sqrt(np.float32(s), dtype=np.float32)
        k = np.round(np.float32(r / c)).astype(np.float32)
        return int(np.sum(np.float32(k * c) >= t))

    out = np.empty(RHO_NUM - 1, np.float32)
    for j in range(1, RHO_NUM):
        lo, hi = 0, int(np.array(1e8, np.float32).view(np.uint32))
        while lo < hi:
            mid = (lo + hi) // 2
            if bucket_id(np.array(mid, np.uint32).view(np.float32)) >= j:
                hi = mid
            else:
                lo = mid + 1
        out[j - 1] = np.array(lo, np.uint32).view(np.float32)
    return out


_S_LIST = [float(v) for v in _bucket_thresholds()]


def _tc_body(z1_ref, z2_ref, col_ref, msk_ref, ind_ref, s_scr, m_scr):
    dx = z1_ref[:, D - 2] - z2_ref[:, D - 2]
    dy = z1_ref[:, D - 1] - z2_ref[:, D - 1]
    s = dx * dx + dy * dy
    mask = (z1_ref[:, 0] > 0.0) & (z2_ref[:, 0] > 0.0)
    sl = _TC_BLK // 128
    s_scr[...] = s.reshape(sl, 128)
    m_scr[...] = jnp.where(mask, 1.0, 0.0).reshape(sl, 128)
    s8 = s_scr[...]
    bid8 = jnp.zeros((sl, 128), jnp.int32)
    for thr in _S_LIST:
        bid8 = bid8 + (s8 >= thr).astype(jnp.int32)
    col_ref[...] = bid8.reshape(1, sl, 128)
    msk_ref[...] = m_scr[...].reshape(1, sl, 128)
    ne = jnp.max(jnp.where(z1_ref[...] != z2_ref[...], 1.0, 0.0))
    ind_ref[...] = jnp.full((1, 1, 128), ne, jnp.float32)


def _tc_stage(z_1, z_2):
    sl = _TC_BLK // 128
    return pl.pallas_call(
        _tc_body,
        grid=(_TC_GRID,),
        in_specs=[
            pl.BlockSpec((_TC_BLK, D), lambda i: (i, 0)),
            pl.BlockSpec((_TC_BLK, D), lambda i: (i, 0)),
        ],
        out_specs=[
            pl.BlockSpec((1, sl, 128), lambda i: (i, 0, 0)),
            pl.BlockSpec((1, sl, 128), lambda i: (i, 0, 0)),
            pl.BlockSpec((1, 1, 128), lambda i: (i, 0, 0)),
        ],
        out_shape=[
            jax.ShapeDtypeStruct((_TC_GRID, sl, 128), jnp.int32),
            jax.ShapeDtypeStruct((_TC_GRID, sl, 128), jnp.float32),
            jax.ShapeDtypeStruct((_TC_GRID, 1, 128), jnp.float32),
        ],
        scratch_shapes=[
            pltpu.VMEM((sl, 128), jnp.float32),
            pltpu.VMEM((sl, 128), jnp.float32),
        ],
    )(z_1, z_2)


def _sc_body(dg_hbm, col_hbm, msk_hbm, ind_hbm, sat_hbm,
             slabv, colv, maskv, valv, indv, sem):
    wid = lax.axis_index("s") * 2 + lax.axis_index("c")
    rbase = wid * _ROWS_PER_W

    pltpu.sync_copy(dg_hbm.at[pl.ds(rbase, _ROWS_PER_W)], slabv)
    pltpu.sync_copy(col_hbm.at[pl.ds(rbase, _ROWS_PER_W)], colv)
    pltpu.sync_copy(msk_hbm.at[pl.ds(rbase, _ROWS_PER_W)], maskv)
    pltpu.sync_copy(ind_hbm, indv)

    ne = jnp.zeros((16,), jnp.float32)
    for k in range(_IND_N // 16):
        ne = jnp.maximum(ne, indv[pl.ds(k * 16, 16)])
    gate = jnp.where(jnp.max(ne) > 0.0, 1.0, 0.0)

    lanes = lax.iota(jnp.int32, 16)

    def group(g, _):
        slc = pl.ds(g * 16, 16)
        rloc = g * 16 + lanes
        cols = colv[slc]
        v = plsc.load_gather(slabv, [rloc, cols])
        valv[slc] = v * maskv[slc] * gate
        return 0

    lax.fori_loop(0, _GROUPS, group, 0)
    pltpu.sync_copy(valv, sat_hbm.at[pl.ds(rbase, _ROWS_PER_W)])


_sc_fn = functools.partial(
    pl.kernel,
    mesh=plsc.VectorSubcoreMesh(core_axis_name="c", subcore_axis_name="s"),
    compiler_params=pltpu.CompilerParams(needs_layout_passes=False),
    out_type=jax.ShapeDtypeStruct((B,), jnp.float32),
    scratch_types=[
        pltpu.VMEM((_ROWS_PER_W, RHO_NUM), jnp.float32),
        pltpu.VMEM((_ROWS_PER_W,), jnp.int32),
        pltpu.VMEM((_ROWS_PER_W,), jnp.float32),
        pltpu.VMEM((_ROWS_PER_W,), jnp.float32),
        pltpu.VMEM((_IND_N,), jnp.float32),
        pltpu.SemaphoreType.DMA,
    ],
)(_sc_body)


def kernel(z_1, z_2, dist_grade, img, given_param):
    col, msk, ind = _tc_stage(z_1, z_2)
    return _sc_fn(dist_grade, col.reshape(-1), msk.reshape(-1),
                  ind.reshape(-1))


# speedup vs baseline: 1.1431x; 1.0254x over previous
import functools

import jax
import jax.numpy as jnp
import numpy as np
from jax import lax
from jax.experimental import pallas as pl
from jax.experimental.pallas import tpu as pltpu
from jax.experimental.pallas import tpu_sc as plsc

RHO_NUM = 100
B = 16384
D = 11

_TC_BLK = 4096
_TC_GRID = B // _TC_BLK
_ROWS_PER_W = B // 32
_GROUPS = _ROWS_PER_W // 16
_IND_N = _TC_GRID * 128


def _bucket_thresholds():
    c = np.float32(1.0 / RHO_NUM)
    t = np.array([np.float32(0.01 * i) for i in range(1, RHO_NUM)], np.float32)

    def bucket_id(s):
        r = np.sqrt(np.float32(s), dtype=np.float32)
        k = np.round(np.float32(r / c)).astype(np.float32)
        return int(np.sum(np.float32(k * c) >= t))

    out = np.empty(RHO_NUM - 1, np.float32)
    for j in range(1, RHO_NUM):
        lo, hi = 0, int(np.array(1e8, np.float32).view(np.uint32))
        while lo < hi:
            mid = (lo + hi) // 2
            if bucket_id(np.array(mid, np.uint32).view(np.float32)) >= j:
                hi = mid
            else:
                lo = mid + 1
        out[j - 1] = np.array(lo, np.uint32).view(np.float32)
    return out


_S_LIST = [float(v) for v in _bucket_thresholds()]


def _tc_body(z1_ref, z2_ref, col_ref, msk_ref, ind_ref, s_scr, m_scr):
    dx = z1_ref[:, D - 2] - z2_ref[:, D - 2]
    dy = z1_ref[:, D - 1] - z2_ref[:, D - 1]
    s = dx * dx + dy * dy
    mask = (z1_ref[:, 0] > 0.0) & (z2_ref[:, 0] > 0.0)
    sl = _TC_BLK // 128
    s_scr[...] = s.reshape(sl, 128)
    m_scr[...] = jnp.where(mask, 1.0, 0.0).reshape(sl, 128)
    s8 = s_scr[...]
    bid8 = jnp.zeros((sl, 128), jnp.int32)
    for thr in _S_LIST:
        bid8 = bid8 + (s8 >= thr).astype(jnp.int32)
    col_ref[...] = bid8.reshape(1, sl, 128)
    msk_ref[...] = m_scr[...].reshape(1, sl, 128)
    ne = jnp.max(jnp.where(z1_ref[...] != z2_ref[...], 1.0, 0.0))
    ind_ref[...] = jnp.full((1, 1, 128), ne, jnp.float32)


def _tc_stage(z_1, z_2):
    sl = _TC_BLK // 128
    return pl.pallas_call(
        _tc_body,
        grid=(_TC_GRID,),
        in_specs=[
            pl.BlockSpec((_TC_BLK, D), lambda i: (i, 0)),
            pl.BlockSpec((_TC_BLK, D), lambda i: (i, 0)),
        ],
        out_specs=[
            pl.BlockSpec((1, sl, 128), lambda i: (i, 0, 0)),
            pl.BlockSpec((1, sl, 128), lambda i: (i, 0, 0)),
            pl.BlockSpec((1, 1, 128), lambda i: (i, 0, 0)),
        ],
        out_shape=[
            jax.ShapeDtypeStruct((_TC_GRID, sl, 128), jnp.int32),
            jax.ShapeDtypeStruct((_TC_GRID, sl, 128), jnp.float32),
            jax.ShapeDtypeStruct((_TC_GRID, 1, 128), jnp.float32),
        ],
        scratch_shapes=[
            pltpu.VMEM((sl, 128), jnp.float32),
            pltpu.VMEM((sl, 128), jnp.float32),
        ],
    )(z_1, z_2)


def _sc_body(dg_hbm, col_hbm, msk_hbm, ind_hbm, sat_hbm,
             slabv, colv, maskv, valv, indv, sem):
    wid = lax.axis_index("s") * 2 + lax.axis_index("c")
    rbase = wid * _ROWS_PER_W

    loads = [
        pltpu.async_copy(dg_hbm.at[pl.ds(rbase, _ROWS_PER_W)], slabv, sem),
        pltpu.async_copy(col_hbm.at[pl.ds(rbase, _ROWS_PER_W)], colv, sem),
        pltpu.async_copy(msk_hbm.at[pl.ds(rbase, _ROWS_PER_W)], maskv, sem),
        pltpu.async_copy(ind_hbm, indv, sem),
    ]
    for ld in loads:
        ld.wait()

    ne = jnp.zeros((16,), jnp.float32)
    for k in range(_IND_N // 16):
        ne = jnp.maximum(ne, indv[pl.ds(k * 16, 16)])
    gate = jnp.where(jnp.max(ne) > 0.0, 1.0, 0.0)

    lanes = lax.iota(jnp.int32, 16)

    def group(g, _):
        slc = pl.ds(g * 16, 16)
        rloc = g * 16 + lanes
        cols = colv[slc]
        v = plsc.load_gather(slabv, [rloc, cols])
        valv[slc] = v * maskv[slc] * gate
        return 0

    lax.fori_loop(0, _GROUPS, group, 0)
    pltpu.sync_copy(valv, sat_hbm.at[pl.ds(rbase, _ROWS_PER_W)])


_sc_fn = functools.partial(
    pl.kernel,
    mesh=plsc.VectorSubcoreMesh(core_axis_name="c", subcore_axis_name="s"),
    compiler_params=pltpu.CompilerParams(needs_layout_passes=False),
    out_type=jax.ShapeDtypeStruct((B,), jnp.float32),
    scratch_types=[
        pltpu.VMEM((_ROWS_PER_W, RHO_NUM), jnp.float32),
        pltpu.VMEM((_ROWS_PER_W,), jnp.int32),
        pltpu.VMEM((_ROWS_PER_W,), jnp.float32),
        pltpu.VMEM((_ROWS_PER_W,), jnp.float32),
        pltpu.VMEM((_IND_N,), jnp.float32),
        pltpu.SemaphoreType.DMA,
    ],
)(_sc_body)


def kernel(z_1, z_2, dist_grade, img, given_param):
    col, msk, ind = _tc_stage(z_1, z_2)
    return _sc_fn(dist_grade, col.reshape(-1), msk.reshape(-1),
                  ind.reshape(-1))
